# baseline (speedup 1.0000x reference)
import functools

import jax
import jax.numpy as jnp
from jax import lax
from jax.experimental import pallas as pl
from jax.experimental.pallas import tpu as pltpu
from jax.experimental.pallas import tpu_sc as plsc

N = 10000
E = 320000
D = 128
EPS = 1e-6

NC = 2
NS = 16
NW = NC * NS
EW = E // NW
CH = 104
K = 3
NFULL = EW // CH
TAIL = EW - NFULL * CH
N_PAD = 10240
RPT = N_PAD // NS
NZ = RPT // CH
ZT = RPT - NZ * CH

_MESH = dict(core_axis_name="c", subcore_axis_name="s")


def _zero_rows(buf, nrows):
    def fill(i, _):
        for j in range(D // 16):
            buf[i, pl.ds(j * 16, 16)] = jnp.zeros((16,), jnp.float32)
        return 0

    lax.fori_loop(0, nrows, fill, 0)


def _zero_acc(zbuf, acc_sh, r0):
    def zspm(j, _):
        pltpu.sync_copy(zbuf, acc_sh.at[pl.ds(r0 + j * CH, CH)])
        return 0

    lax.fori_loop(0, NZ, zspm, 0)
    pltpu.sync_copy(zbuf.at[pl.ds(0, ZT)], acc_sh.at[pl.ds(r0 + NZ * CH, ZT)])


def _agg_pipe(h, src, dst, acc_sh, src_v, dst_v, rows, ss, sd, gs, ebase):
    def issue_loads(g, b):
        off = ebase + g * CH
        pltpu.async_copy(src.at[pl.ds(off, CH)], src_v.at[b], ss[b])
        pltpu.async_copy(dst.at[pl.ds(off, CH)], dst_v.at[b], sd[b])

    def start_gather(b):
        pltpu.make_async_copy(src.at[pl.ds(0, CH)], src_v.at[b],
                              ss[b]).wait()
        pltpu.async_copy(h.at[src_v.at[b]], rows[b], gs[b])

    def finish_scatter(b):
        pltpu.make_async_copy(h.at[src_v.at[b]], rows[b], gs[b]).wait()
        pltpu.make_async_copy(dst.at[pl.ds(0, CH)], dst_v.at[b],
                              sd[b]).wait()
        pltpu.sync_copy(rows[b], acc_sh.at[dst_v.at[b]], add=True)

    for b in range(K):
        issue_loads(b, b)
    for b in range(K):
        start_gather(b)

    def pipe(i, _):
        g0 = K * i
        for b in range(K):
            finish_scatter(b)
            issue_loads(g0 + K + b, b)
            start_gather(b)
        return 0

    lax.fori_loop(0, NFULL // K - 1, pipe, 0)
    for b in range(K):
        finish_scatter(b)


def _agg_tail(h, src, dst, acc_sh, src_t, dst_t, rows_t, st, ebase):
    offt = ebase + NFULL * CH
    pltpu.sync_copy(src.at[pl.ds(offt, TAIL)], src_t)
    pltpu.sync_copy(dst.at[pl.ds(offt, TAIL)], dst_t.at[0])
    pltpu.async_copy(h.at[src_t], rows_t, st).wait()
    pltpu.sync_copy(rows_t, acc_sh.at[dst_t.at[0]], add=True)


def _deg_pipe(dst, acc_sh, dst_v, dst_t, ones, sd, ebase):
    def load_dst(g, b):
        pltpu.async_copy(dst.at[pl.ds(ebase + g * CH, CH)], dst_v.at[b],
                         sd[b])

    def deg_scatter(b):
        pltpu.make_async_copy(dst.at[pl.ds(0, CH)], dst_v.at[b],
                              sd[b]).wait()
        pltpu.sync_copy(ones, acc_sh.at[dst_v.at[b]], add=True)

    load_dst(0, 0)
    load_dst(1, 1)

    def pipe(i, _):
        g0 = 2 * i
        deg_scatter(0)
        load_dst(g0 + 2, 0)
        deg_scatter(1)
        load_dst(g0 + 3, 1)
        return 0

    lax.fori_loop(0, NFULL // 2 - 1, pipe, 0)
    deg_scatter(0)
    deg_scatter(1)
    offt = ebase + NFULL * CH
    pltpu.sync_copy(dst.at[pl.ds(offt, TAIL)], dst_t.at[0])
    pltpu.sync_copy(ones.at[pl.ds(0, TAIL)], acc_sh.at[dst_t.at[0]],
                    add=True)


def _agg_deg_body(h, src, dst, agg_out, deg_out, src_v, dst_v, src_t, dst_t,
                  r0b, r1b, r2b, rows_t, acc_sh, ss0, sd0, gs0, ss1, sd1,
                  gs1, ss2, sd2, gs2, st):
    cid = lax.axis_index("c")
    sid = lax.axis_index("s")
    wid = sid * NC + cid
    r0 = sid * RPT
    ebase = wid * EW
    rows = (r0b, r1b, r2b)
    ss, sd, gs = (ss0, ss1, ss2), (sd0, sd1, sd2), (gs0, gs1, gs2)

    _zero_rows(r0b, CH)
    _zero_acc(r0b, acc_sh, r0)

    def fill1(i, _):
        for j in range(D // 16):
            r0b[i, pl.ds(j * 16, 16)] = jnp.ones((16,), jnp.float32)
        return 0

    lax.fori_loop(0, CH, fill1, 0)
    plsc.subcore_barrier()
    _deg_pipe(dst, acc_sh, dst_v, dst_t, r0b, sd, ebase)
    plsc.subcore_barrier()
    pltpu.sync_copy(acc_sh.at[pl.ds(r0, RPT)], deg_out.at[cid, pl.ds(r0, RPT)])

    _zero_rows(r1b, CH)
    _zero_acc(r1b, acc_sh, r0)
    plsc.subcore_barrier()

    _agg_pipe(h, src, dst, acc_sh, src_v, dst_v, rows, ss, sd, gs, ebase)
    _agg_tail(h, src, dst, acc_sh, src_t, dst_t, rows_t, st, ebase)
    plsc.subcore_barrier()
    pltpu.sync_copy(acc_sh.at[pl.ds(r0, RPT)], agg_out.at[cid, pl.ds(r0, RPT)])


def _agg_body(h, src, dst, agg_out, src_v, dst_v, src_t, dst_t, r0b, r1b,
              r2b, rows_t, acc_sh, ss0, sd0, gs0, ss1, sd1, gs1, ss2, sd2,
              gs2, st):
    cid = lax.axis_index("c")
    sid = lax.axis_index("s")
    wid = sid * NC + cid
    r0 = sid * RPT
    ebase = wid * EW
    rows = (r0b, r1b, r2b)
    ss, sd, gs = (ss0, ss1, ss2), (sd0, sd1, sd2), (gs0, gs1, gs2)

    _zero_rows(r0b, CH)
    _zero_acc(r0b, acc_sh, r0)
    plsc.subcore_barrier()

    _agg_pipe(h, src, dst, acc_sh, src_v, dst_v, rows, ss, sd, gs, ebase)
    _agg_tail(h, src, dst, acc_sh, src_t, dst_t, rows_t, st, ebase)
    plsc.subcore_barrier()
    pltpu.sync_copy(acc_sh.at[pl.ds(r0, RPT)], agg_out.at[cid, pl.ds(r0, RPT)])


_SC_SCRATCH = [
    pltpu.VMEM((K, CH), jnp.int32),
    pltpu.VMEM((K, CH), jnp.int32),
    pltpu.VMEM((TAIL,), jnp.int32),
    pltpu.VMEM((1, TAIL), jnp.int32),
    pltpu.VMEM((CH, D), jnp.float32),
    pltpu.VMEM((CH, D), jnp.float32),
    pltpu.VMEM((CH, D), jnp.float32),
    pltpu.VMEM((TAIL, D), jnp.float32),
    pltpu.VMEM_SHARED((N_PAD, D), jnp.float32),
] + [pltpu.SemaphoreType.DMA] * 10


def _make_agg_deg():
    return functools.partial(
        pl.kernel,
        mesh=plsc.VectorSubcoreMesh(**_MESH),
        out_type=[jax.ShapeDtypeStruct((NC, N_PAD, D), jnp.float32),
                  jax.ShapeDtypeStruct((NC, N_PAD, D), jnp.float32)],
        scratch_types=list(_SC_SCRATCH),
    )(_agg_deg_body)


def _make_agg():
    return functools.partial(
        pl.kernel,
        mesh=plsc.VectorSubcoreMesh(**_MESH),
        out_type=jax.ShapeDtypeStruct((NC, N_PAD, D), jnp.float32),
        scratch_types=list(_SC_SCRATCH),
    )(_agg_body)


def _logmap_body(x_ref, o_ref):
    v = x_ref[...]
    n = jnp.sqrt(jnp.sum(v * v, axis=1, keepdims=True))
    nc = jnp.clip(n, EPS, 1.0 - 1e-5)
    o_ref[...] = (0.5 * jnp.log((1.0 + nc) / (1.0 - nc))) * v / nc


def _tc_logmap(x):
    blk = 1000
    return pl.pallas_call(
        _logmap_body,
        out_shape=jax.ShapeDtypeStruct((N, D), jnp.float32),
        grid=(N // blk,),
        in_specs=[pl.BlockSpec((blk, D), lambda i: (i, 0))],
        out_specs=pl.BlockSpec((blk, D), lambda i: (i, 0)),
    )(x)


def _combine_body(agg_ref, deg_ref, w_ref, o_ref, *, last):
    a = agg_ref[0] + agg_ref[1]
    d = jnp.sum(deg_ref[...], axis=(0, 2)) * (1.0 / D)
    a = a / jnp.clip(d, 1.0, None)[:, None]
    out = jnp.dot(a, w_ref[...], preferred_element_type=jnp.float32)
    n = jnp.sqrt(jnp.sum(out * out, axis=1, keepdims=True))
    nc = jnp.clip(n, EPS, None)
    y = jnp.tanh(nc) * out / nc
    if not last:
        m = jnp.sqrt(jnp.sum(y * y, axis=1, keepdims=True))
        mc = jnp.clip(m, EPS, 1.0 - 1e-5)
        y = (0.5 * jnp.log((1.0 + mc) / (1.0 - mc))) * y / mc
    o_ref[...] = y


def _tc_combine(agg, deg, w, last):
    blk = 1024
    return pl.pallas_call(
        functools.partial(_combine_body, last=last),
        out_shape=jax.ShapeDtypeStruct((N_PAD, D), jnp.float32),
        grid=(N_PAD // blk,),
        in_specs=[
            pl.BlockSpec((NC, blk, D), lambda i: (0, i, 0)),
            pl.BlockSpec((NC, blk, D), lambda i: (0, i, 0)),
            pl.BlockSpec((D, D), lambda i: (0, 0)),
        ],
        out_specs=pl.BlockSpec((blk, D), lambda i: (i, 0)),
    )(agg, deg, w)


def kernel(x, edge_index, W0, W1):
    src = edge_index[0].astype(jnp.int32)
    dst = edge_index[1].astype(jnp.int32)
    h0 = _tc_logmap(x)
    agg1, deg = _make_agg_deg()(h0, src, dst)
    h1 = _tc_combine(agg1, deg, W0, last=False)
    agg2 = _make_agg()(h1, src, dst)
    y = _tc_combine(agg2, deg, W1, last=True)
    return y[:N]
